# baseline (speedup 1.0000x reference)
import functools

import jax
import jax.numpy as jnp
from jax import lax
from jax.experimental import pallas as pl
from jax.experimental.pallas import tpu as pltpu
from jax.experimental.pallas import tpu_sc as plsc


_BS = 512


def _tc_add_kernel(x_ref, pos_ref, o_ref):
    o_ref[...] = x_ref[...] + pos_ref[...][None, :, :]


def _tc_kernel(x, pos_table):
    batch, seq, d = x.shape
    bs = _BS if seq % _BS == 0 else seq
    grid = (seq // bs,)
    return pl.pallas_call(
        _tc_add_kernel,
        grid=grid,
        in_specs=[
            pl.BlockSpec((batch, bs, d), lambda i: (0, i, 0)),
            pl.BlockSpec((bs, d), lambda i: (i, 0)),
        ],
        out_specs=pl.BlockSpec((batch, bs, d), lambda i: (0, i, 0)),
        out_shape=jax.ShapeDtypeStruct((batch, seq, d), x.dtype),
    )(x, pos_table)


_NW = 32
_CH_ROWS = 64


def _sc_kernel(x, pos_table):
    batch, seq, d = x.shape
    nrows = batch * seq
    rows_per_w = nrows // _NW
    nchunks = rows_per_w // _CH_ROWS
    chunk = _CH_ROWS * d
    nvreg = chunk // 16

    mesh = plsc.VectorSubcoreMesh(core_axis_name="c", subcore_axis_name="s")

    @functools.partial(
        pl.kernel,
        mesh=mesh,
        out_type=jax.ShapeDtypeStruct((nrows * d,), jnp.float32),
        scratch_types=[
            pltpu.VMEM((chunk,), jnp.float32),
            pltpu.VMEM((chunk,), jnp.float32),
        ],
    )
    def k(x_hbm, pos_hbm, o_hbm, xbuf, pbuf):
        wid = lax.axis_index("s") * 2 + lax.axis_index("c")
        base = wid * rows_per_w
        pbase = lax.rem(base, seq)

        def do_chunk(i, _):
            r0 = (base + i * _CH_ROWS) * d
            p0 = (pbase + i * _CH_ROWS) * d
            pltpu.sync_copy(x_hbm.at[pl.ds(r0, chunk)], xbuf)
            pltpu.sync_copy(pos_hbm.at[pl.ds(p0, chunk)], pbuf)

            def add(kk, _):
                o = kk * 16
                xbuf[pl.ds(o, 16)] = xbuf[pl.ds(o, 16)] + pbuf[pl.ds(o, 16)]
                return 0

            lax.fori_loop(0, nvreg, add, 0)
            pltpu.sync_copy(xbuf, o_hbm.at[pl.ds(r0, chunk)])
            return 0

        lax.fori_loop(0, nchunks, do_chunk, 0)

    out = k(x.reshape(-1), pos_table.reshape(-1))
    return out.reshape(batch, seq, d)


def kernel(x, pos_table):
    return _sc_kernel(x, pos_table)


# speedup vs baseline: 8.2907x; 8.2907x over previous
import functools

import jax
import jax.numpy as jnp
from jax import lax
from jax.experimental import pallas as pl
from jax.experimental.pallas import tpu as pltpu
from jax.experimental.pallas import tpu_sc as plsc


_BS = 1024


def _tc_add_kernel(x_ref, pos_ref, o_ref):
    o_ref[...] = x_ref[...] + pos_ref[...][None, :, :]


def _tc_kernel(x, pos_table):
    batch, seq, d = x.shape
    bs = _BS if seq % _BS == 0 else seq
    grid = (seq // bs,)
    return pl.pallas_call(
        _tc_add_kernel,
        grid=grid,
        in_specs=[
            pl.BlockSpec((batch, bs, d), lambda i: (0, i, 0)),
            pl.BlockSpec((bs, d), lambda i: (i, 0)),
        ],
        out_specs=pl.BlockSpec((batch, bs, d), lambda i: (0, i, 0)),
        out_shape=jax.ShapeDtypeStruct((batch, seq, d), x.dtype),
    )(x, pos_table)


_NW = 32
_CH_ROWS = 64


def _sc_kernel(x, pos_table):
    batch, seq, d = x.shape
    nrows = batch * seq
    rows_per_w = nrows // _NW
    nchunks = rows_per_w // _CH_ROWS
    chunk = _CH_ROWS * d
    nvreg = chunk // 16

    mesh = plsc.VectorSubcoreMesh(core_axis_name="c", subcore_axis_name="s")

    @functools.partial(
        pl.kernel,
        mesh=mesh,
        out_type=jax.ShapeDtypeStruct((nrows * d,), jnp.float32),
        scratch_types=[
            pltpu.VMEM((chunk,), jnp.float32),
            pltpu.VMEM((chunk,), jnp.float32),
        ],
    )
    def k(x_hbm, pos_hbm, o_hbm, xbuf, pbuf):
        wid = lax.axis_index("s") * 2 + lax.axis_index("c")
        base = wid * rows_per_w
        pbase = lax.rem(base, seq)

        def do_chunk(i, _):
            r0 = (base + i * _CH_ROWS) * d
            p0 = (pbase + i * _CH_ROWS) * d
            pltpu.sync_copy(x_hbm.at[pl.ds(r0, chunk)], xbuf)
            pltpu.sync_copy(pos_hbm.at[pl.ds(p0, chunk)], pbuf)

            def add(kk, _):
                o = kk * 16
                xbuf[pl.ds(o, 16)] = xbuf[pl.ds(o, 16)] + pbuf[pl.ds(o, 16)]
                return 0

            lax.fori_loop(0, nvreg, add, 0)
            pltpu.sync_copy(xbuf, o_hbm.at[pl.ds(r0, chunk)])
            return 0

        lax.fori_loop(0, nchunks, do_chunk, 0)

    out = k(x.reshape(-1), pos_table.reshape(-1))
    return out.reshape(batch, seq, d)


def kernel(x, pos_table):
    return _tc_kernel(x, pos_table)
